# baseline (speedup 1.0000x reference)
import jax
import jax.numpy as jnp
from jax import lax
from jax.experimental import pallas as pl
from jax.experimental.pallas import tpu as pltpu
from jax.experimental.pallas import tpu_sc as plsc

NUM_EMB = 1000000
DIM = 32
BATCH = 16384
HIST = 50

_info = plsc.get_sparse_core_info()
NC, NS = _info.num_cores, _info.num_subcores
NW = NC * NS
B_PER_W = BATCH // NW
G = 128
NGPL = B_PER_W // G


def _body(table_hbm, xt_hbm, out_hbm, idx_all,
          rows0, rows1, tb0, tb1, gsem0, gsem1, osem0, osem1):
    wid = lax.axis_index("s") * NC + lax.axis_index("c")
    bbase = wid * B_PER_W
    lanes = lax.iota(jnp.int32, 16)
    cols = [jnp.full((16,), dd, jnp.int32) for dd in range(DIM)]

    pltpu.sync_copy(xt_hbm.at[:, pl.ds(bbase, B_PER_W)], idx_all)

    def fire(l, rows, sem):
        pltpu.async_copy(table_hbm.at[idx_all.at[l]], rows, sem)

    def drain(rows, sem):
        pltpu.make_async_copy(
            table_hbm.at[idx_all.at[0]], rows, sem).wait()

    def transpose(rows, tb):
        @plsc.parallel_loop(0, B_PER_W, unroll=8)
        def tr(j):
            jsplat = jnp.full((16,), 0, jnp.int32) + j
            lo = rows[j, pl.ds(0, 16)]
            hi = rows[j, pl.ds(16, 16)]
            plsc.store_scatter(tb, [lanes, jsplat], lo)
            plsc.store_scatter(tb, [lanes + 16, jsplat], hi)

    def out_slice(l):
        return out_hbm.at[l].at[:, pl.ds(bbase, B_PER_W)]

    def wait_out(tb, sem):
        pltpu.make_async_copy(tb, out_slice(0), sem).wait()

    fire(0, rows0, gsem0)

    def step(i, _):
        l = 2 * i

        @pl.when(i > 0)
        def _():
            wait_out(tb0, osem0)
        fire(l + 1, rows1, gsem1)
        drain(rows0, gsem0)
        transpose(rows0, tb0)
        pltpu.async_copy(tb0, out_slice(l), osem0)

        @pl.when(i > 0)
        def _():
            wait_out(tb1, osem1)
        @pl.when(i < HIST // 2 - 1)
        def _():
            fire(l + 2, rows0, gsem0)
        drain(rows1, gsem1)
        transpose(rows1, tb1)
        pltpu.async_copy(tb1, out_slice(l + 1), osem1)
        return ()

    lax.fori_loop(0, HIST // 2, step, ())
    wait_out(tb0, osem0)
    wait_out(tb1, osem1)


@jax.jit
def _gather_t(table, xt):
    mesh = plsc.VectorSubcoreMesh(core_axis_name="c", subcore_axis_name="s")
    return pl.kernel(
        _body,
        out_type=jax.ShapeDtypeStruct((HIST, DIM, BATCH), jnp.float32),
        mesh=mesh,
        scratch_types=[
            pltpu.VMEM((HIST, B_PER_W), jnp.int32),
            pltpu.VMEM((B_PER_W, DIM), jnp.float32),
            pltpu.VMEM((B_PER_W, DIM), jnp.float32),
            pltpu.VMEM((DIM, B_PER_W), jnp.float32),
            pltpu.VMEM((DIM, B_PER_W), jnp.float32),
            pltpu.SemaphoreType.DMA,
            pltpu.SemaphoreType.DMA,
            pltpu.SemaphoreType.DMA,
            pltpu.SemaphoreType.DMA,
        ],
        compiler_params=pltpu.CompilerParams(
            use_tc_tiling_on_sc=False, needs_layout_passes=False),
    )(table, xt)


def kernel(x, table):
    xt = x.T.astype(jnp.int32)
    out_t = _gather_t(table, xt)
    return jnp.transpose(out_t, (2, 0, 1))


# speedup vs baseline: 1.4871x; 1.4871x over previous
import jax
import jax.numpy as jnp
from jax import lax
from jax.experimental import pallas as pl
from jax.experimental.pallas import tpu as pltpu
from jax.experimental.pallas import tpu_sc as plsc

NUM_EMB = 1000000
DIM = 32
BATCH = 16384
HIST = 50

_info = plsc.get_sparse_core_info()
NC, NS = _info.num_cores, _info.num_subcores
NW = NC * NS
B_PER_W = BATCH // NW
G = 128
NGPL = B_PER_W // G


def _body(table_hbm, xt_hbm, out_hbm, idx_all,
          rows0, rows1, tb0, tb1, gsem0, gsem1, osem0, osem1):
    wid = lax.axis_index("s") * NC + lax.axis_index("c")
    bbase = wid * B_PER_W
    lanes = lax.iota(jnp.int32, 16)
    cols = [jnp.full((16,), dd, jnp.int32) for dd in range(DIM)]

    pltpu.sync_copy(xt_hbm.at[:, pl.ds(bbase, B_PER_W)], idx_all)

    def fire(l, rows, sem):
        pltpu.async_copy(table_hbm.at[idx_all.at[l]], rows, sem)

    def drain(rows, sem):
        pltpu.make_async_copy(
            table_hbm.at[idx_all.at[0]], rows, sem).wait()

    def transpose(rows, tb):
        @plsc.parallel_loop(0, B_PER_W, unroll=8)
        def tr(j):
            jsplat = jnp.full((16,), 0, jnp.int32) + j
            lo = rows[j, pl.ds(0, 16)]
            hi = rows[j, pl.ds(16, 16)]
            plsc.store_scatter(tb, [lanes, jsplat], lo)
            plsc.store_scatter(tb, [lanes + 16, jsplat], hi)

    def out_slice(l):
        return out_hbm.at[l].at[:, pl.ds(bbase, B_PER_W)]

    def wait_out(tb, sem):
        pltpu.make_async_copy(
            tb.at[:, pl.ds(0, B_PER_W)], out_slice(0), sem).wait()

    fire(0, rows0, gsem0)

    def step(i, _):
        l = 2 * i

        @pl.when(i > 0)
        def _():
            wait_out(tb0, osem0)
        fire(l + 1, rows1, gsem1)
        drain(rows0, gsem0)
        transpose(rows0, tb0)
        pltpu.async_copy(tb0.at[:, pl.ds(0, B_PER_W)], out_slice(l), osem0)

        @pl.when(i > 0)
        def _():
            wait_out(tb1, osem1)
        @pl.when(i < HIST // 2 - 1)
        def _():
            fire(l + 2, rows0, gsem0)
        drain(rows1, gsem1)
        transpose(rows1, tb1)
        pltpu.async_copy(tb1.at[:, pl.ds(0, B_PER_W)], out_slice(l + 1), osem1)
        return ()

    lax.fori_loop(0, HIST // 2, step, ())
    wait_out(tb0, osem0)
    wait_out(tb1, osem1)


@jax.jit
def _gather_t(table, xt):
    mesh = plsc.VectorSubcoreMesh(core_axis_name="c", subcore_axis_name="s")
    return pl.kernel(
        _body,
        out_type=jax.ShapeDtypeStruct((HIST, DIM, BATCH), jnp.float32),
        mesh=mesh,
        scratch_types=[
            pltpu.VMEM((HIST, B_PER_W), jnp.int32),
            pltpu.VMEM((B_PER_W, DIM), jnp.float32),
            pltpu.VMEM((B_PER_W, DIM), jnp.float32),
            pltpu.VMEM((DIM, B_PER_W + 1), jnp.float32),
            pltpu.VMEM((DIM, B_PER_W + 1), jnp.float32),
            pltpu.SemaphoreType.DMA,
            pltpu.SemaphoreType.DMA,
            pltpu.SemaphoreType.DMA,
            pltpu.SemaphoreType.DMA,
        ],
        compiler_params=pltpu.CompilerParams(
            use_tc_tiling_on_sc=False, needs_layout_passes=False),
    )(table, xt)


def kernel(x, table):
    xt = x.T.astype(jnp.int32)
    out_t = _gather_t(table, xt)
    return jnp.transpose(out_t, (2, 0, 1))
